# baseline (speedup 1.0000x reference)
import jax
import jax.numpy as jnp
from jax import lax
from jax.experimental import pallas as pl
from jax.experimental.pallas import tpu as pltpu
from jax.experimental.pallas import tpu_sc as plsc

_VOCAB = 1000000
_EMBED_DIM = 64
_BATCH = 16384

_info = plsc.get_sparse_core_info()
_NC = _info.num_cores
_NS = _info.num_subcores
_NW = _NC * _NS
_L = 16

_CHUNK = 256
_NFULL = 3906
_NCH = 3907
_RB = 64
_NBUF = 6


def _body(idx_hbm, wt_hbm, out_hbm, islab, lpk, staged, clane, cpos,
          rbuf, subpos, sem):
    wid = lax.axis_index("s") * _NC + lax.axis_index("c")
    lanes = lax.iota(jnp.int32, _L)
    dummy = _BATCH + wid
    zeros = jnp.zeros((_L,), jnp.int32)
    n_o = (_NCH // _NW) + jnp.where(wid < _NCH % _NW, 1, 0)

    def enqueue(o, p):
        c = wid + _NW * o

        def tail_fetch():
            col0 = pl.multiple_of(jnp.minimum(c, _NFULL) * _CHUNK, 128)
            pltpu.async_copy(
                wt_hbm.at[:, pl.ds(col0, 128)],
                staged.at[p].at[:, pl.ds(0, 128)], sem)
            return 0

        def full_fetch():
            col0 = pl.multiple_of(jnp.minimum(c, _NFULL - 1) * _CHUNK, 128)
            pltpu.async_copy(
                wt_hbm.at[:, pl.ds(col0, _CHUNK)], staged.at[p], sem)
            return 0

        lax.cond(c == _NFULL, tail_fetch, full_fetch)

    def wait_fetch(o):
        c = wid + _NW * o

        def tail_wait():
            pltpu.make_async_copy(
                wt_hbm.at[:, pl.ds(0, 128)],
                staged.at[0].at[:, pl.ds(0, 128)], sem).wait()
            return 0

        def full_wait():
            pltpu.make_async_copy(
                wt_hbm.at[:, pl.ds(0, _CHUNK)], staged.at[0], sem).wait()
            return 0

        lax.cond(c == _NFULL, tail_wait, full_wait)

    clane[pl.ds(0, _L)] = zeros
    clane[pl.ds(_L, _L)] = zeros
    cpos[pl.ds(0, _L)] = zeros
    cpos[pl.ds(_L, _L)] = zeros

    def reset_subpos():
        dv = jnp.full((_L,), dummy, jnp.int32)
        for t in range(_RB // _L):
            subpos[pl.ds(t * _L, _L)] = dv

    reset_subpos()

    for k in range(_NBUF - 1):
        enqueue(k, k)

    def slab_body(s, cnt):
        pltpu.sync_copy(
            idx_hbm.at[pl.ds(pl.multiple_of(s * _L, _L), _L)], islab)

        def p1_body(g, cnt):
            r = lax.shift_right_logical(g, 3)
            c0 = pl.multiple_of((g & 7) * _L, _L)
            vi = islab[r, pl.ds(c0, _L)]
            chv = jnp.minimum(lax.shift_right_logical(vi, 8), _NCH - 1)
            m = (chv & (_NW - 1)) == wid
            mi = m.astype(jnp.int32)
            rank = plsc.cumsum(mi) - mi
            dest = cnt + rank
            lane = vi - chv * _CHUNK
            posv = (s * 2048 + g * _L) + lanes
            packed = (
                lax.shift_left(lax.shift_right_logical(chv, 5), 24)
                | lax.shift_left(lane, 14) | posv
            )
            plsc.store_scatter(lpk, [dest], packed, mask=m)
            return cnt + jnp.sum(mi)

        return lax.fori_loop(0, 128, p1_body, cnt)

    nloc = lax.fori_loop(0, _BATCH // (_L * 128), slab_body, 0)
    ngr = lax.shift_right_logical(nloc + _L - 1, 4)

    def flush(cb):
        pltpu.sync_copy(rbuf, out_hbm.at[subpos])
        reset_subpos()
        return 0 * cb

    def drain(nv, cb, pf):
        lanev = clane[pl.ds(0, _L)]
        posv = cpos[pl.ds(0, _L)]
        vmask = lanes < nv
        destrow = cb + lanes
        for comp in range(_EMBED_DIM):
            cf = jnp.full((_L,), comp, jnp.int32)
            vals = plsc.load_gather(staged, [pf, cf, lanev])
            plsc.store_scatter(rbuf, [destrow, cf], vals, mask=vmask)
        plsc.store_scatter(subpos, [destrow],
                           jnp.where(vmask, posv, dummy))
        clane[pl.ds(0, _L)] = clane[pl.ds(_L, _L)]
        cpos[pl.ds(0, _L)] = cpos[pl.ds(_L, _L)]
        cb2 = cb + nv
        return lax.cond(cb2 > _RB - _L, lambda: flush(cb2), lambda: cb2)

    def chunk_body(o, carry):
        cb, p = carry
        p2 = jnp.where(p == 0, _NBUF - 1, p - 1)
        pl.when(o + _NBUF - 1 < n_o)(lambda: enqueue(o + _NBUF - 1, p2))
        wait_fetch(o)
        pf = jnp.full((_L,), p, jnp.int32)

        def scan_body(j, carry):
            cs, cb = carry
            off = pl.multiple_of(j * _L, _L)
            lv = lpk[pl.ds(off, _L)]
            m = (lax.shift_right_logical(lv, 24) == o) \
                & ((j * _L + lanes) < nloc)
            mi = m.astype(jnp.int32)
            rank = plsc.cumsum(mi) - mi
            dest = cs + rank
            plsc.store_scatter(
                clane, [dest],
                lax.shift_right_logical(lv, 14) & 1023, mask=m)
            plsc.store_scatter(cpos, [dest], lv & 16383, mask=m)
            cs = cs + jnp.sum(mi)
            cb = lax.cond(cs >= _L, lambda: drain(_L, cb, pf), lambda: cb)
            cs = lax.cond(cs >= _L, lambda: cs - _L, lambda: cs)
            return (cs, cb)

        cs, cb = lax.fori_loop(0, ngr, scan_body, (0, cb))
        cb = lax.cond(cs > 0, lambda: drain(cs, cb, pf), lambda: cb)
        pn = jnp.where(p == _NBUF - 1, 0, p + 1)
        return (cb, pn)

    cb, _ = lax.fori_loop(0, n_o, chunk_body, (0, 0))
    lax.cond(cb > 0, lambda: flush(cb), lambda: cb)


@jax.jit
def kernel(input_, weight):
    idx2 = input_.astype(jnp.int32).reshape(_BATCH // 128, 128)
    wt = jnp.swapaxes(weight, 0, 1)
    f = pl.kernel(
        _body,
        mesh=plsc.VectorSubcoreMesh(core_axis_name="c", subcore_axis_name="s"),
        out_type=jax.ShapeDtypeStruct((_BATCH + _NW, 128), jnp.float32),
        scratch_types=[
            pltpu.VMEM((_L, 128), jnp.int32),
            pltpu.VMEM((_BATCH,), jnp.int32),
            pltpu.VMEM((_NBUF, _EMBED_DIM, _CHUNK), jnp.float32),
            pltpu.VMEM((2 * _L,), jnp.int32),
            pltpu.VMEM((2 * _L,), jnp.int32),
            pltpu.VMEM((_RB, 128), jnp.float32),
            pltpu.VMEM((_RB,), jnp.int32),
            pltpu.SemaphoreType.DMA,
        ],
        compiler_params=pltpu.CompilerParams(needs_layout_passes=False),
    )
    out2 = f(idx2, wt)
    return out2[:_BATCH, :_EMBED_DIM]


# speedup vs baseline: 1.4344x; 1.4344x over previous
import jax
import jax.numpy as jnp
from jax import lax
from jax.experimental import pallas as pl
from jax.experimental.pallas import tpu as pltpu
from jax.experimental.pallas import tpu_sc as plsc

_VOCAB = 1000000
_EMBED_DIM = 64
_BATCH = 16384

_info = plsc.get_sparse_core_info()
_NC = _info.num_cores
_NS = _info.num_subcores
_NW = _NC * _NS
_L = 16

_CHUNK = 512
_NFULL = 1953
_NCH = 1954
_RB = 64
_NBUF = 3


def _body(idx_hbm, wt_hbm, out_hbm, islab, lpk, staged, clane, cpos,
          rbuf, subpos, sem):
    wid = lax.axis_index("s") * _NC + lax.axis_index("c")
    lanes = lax.iota(jnp.int32, _L)
    dummy = _BATCH + wid
    zeros = jnp.zeros((_L,), jnp.int32)
    n_o = (_NCH // _NW) + jnp.where(wid < _NCH % _NW, 1, 0)

    def enqueue(o, p):
        c = wid + _NW * o

        def tail_fetch():
            col0 = pl.multiple_of(jnp.minimum(c, _NFULL) * _CHUNK, 128)
            pltpu.async_copy(
                wt_hbm.at[:, pl.ds(col0, 128)],
                staged.at[p].at[:, pl.ds(0, 128)], sem)
            return 0

        def full_fetch():
            col0 = pl.multiple_of(jnp.minimum(c, _NFULL - 1) * _CHUNK, 128)
            pltpu.async_copy(
                wt_hbm.at[:, pl.ds(col0, _CHUNK)], staged.at[p], sem)
            return 0

        lax.cond(c == _NFULL, tail_fetch, full_fetch)

    def wait_fetch(o):
        c = wid + _NW * o

        def tail_wait():
            pltpu.make_async_copy(
                wt_hbm.at[:, pl.ds(0, 128)],
                staged.at[0].at[:, pl.ds(0, 128)], sem).wait()
            return 0

        def full_wait():
            pltpu.make_async_copy(
                wt_hbm.at[:, pl.ds(0, _CHUNK)], staged.at[0], sem).wait()
            return 0

        lax.cond(c == _NFULL, tail_wait, full_wait)

    clane[pl.ds(0, _L)] = zeros
    clane[pl.ds(_L, _L)] = zeros
    cpos[pl.ds(0, _L)] = zeros
    cpos[pl.ds(_L, _L)] = zeros

    def reset_subpos():
        dv = jnp.full((_L,), dummy, jnp.int32)
        for t in range(_RB // _L):
            subpos[pl.ds(t * _L, _L)] = dv

    reset_subpos()

    enqueue(0, 0)
    enqueue(1, 1)

    def slab_body(s, cnt):
        pltpu.sync_copy(
            idx_hbm.at[pl.ds(pl.multiple_of(s * _L, _L), _L)], islab)

        def p1_body(g, cnt):
            r = lax.shift_right_logical(g, 3)
            c0 = pl.multiple_of((g & 7) * _L, _L)
            vi = islab[r, pl.ds(c0, _L)]
            chv = jnp.minimum(lax.shift_right_logical(vi, 9), _NCH - 1)
            m = (chv & (_NW - 1)) == wid
            mi = m.astype(jnp.int32)
            rank = plsc.cumsum(mi) - mi
            dest = cnt + rank
            lane = vi - chv * _CHUNK
            posv = (s * 2048 + g * _L) + lanes
            packed = (
                lax.shift_left(lax.shift_right_logical(chv, 5), 24)
                | lax.shift_left(lane, 14) | posv
            )
            plsc.store_scatter(lpk, [dest], packed, mask=m)
            return cnt + jnp.sum(mi)

        return lax.fori_loop(0, 128, p1_body, cnt)

    nloc = lax.fori_loop(0, _BATCH // (_L * 128), slab_body, 0)
    ngr = lax.shift_right_logical(nloc + _L - 1, 4)

    def flush(cb):
        pltpu.sync_copy(rbuf, out_hbm.at[subpos])
        reset_subpos()
        return 0 * cb

    def drain(nv, cb, pf):
        lanev = clane[pl.ds(0, _L)]
        posv = cpos[pl.ds(0, _L)]
        vmask = lanes < nv
        destrow = cb + lanes
        for comp in range(_EMBED_DIM):
            cf = jnp.full((_L,), comp, jnp.int32)
            vals = plsc.load_gather(staged, [pf, cf, lanev])
            plsc.store_scatter(rbuf, [destrow, cf], vals, mask=vmask)
        plsc.store_scatter(subpos, [destrow],
                           jnp.where(vmask, posv, dummy))
        clane[pl.ds(0, _L)] = clane[pl.ds(_L, _L)]
        cpos[pl.ds(0, _L)] = cpos[pl.ds(_L, _L)]
        cb2 = cb + nv
        return lax.cond(cb2 > _RB - _L, lambda: flush(cb2), lambda: cb2)

    def chunk_body(o, carry):
        cb, p = carry
        p2 = jnp.where(p == 0, _NBUF - 1, p - 1)
        pl.when(o + 2 < n_o)(lambda: enqueue(o + 2, p2))
        wait_fetch(o)
        pf = jnp.full((_L,), p, jnp.int32)

        def scan_body(j, carry):
            cs, cb = carry
            off = pl.multiple_of(j * _L, _L)
            lv = lpk[pl.ds(off, _L)]
            m = (lax.shift_right_logical(lv, 24) == o) \
                & ((j * _L + lanes) < nloc)
            mi = m.astype(jnp.int32)
            rank = plsc.cumsum(mi) - mi
            dest = cs + rank
            plsc.store_scatter(
                clane, [dest],
                lax.shift_right_logical(lv, 14) & 1023, mask=m)
            plsc.store_scatter(cpos, [dest], lv & 16383, mask=m)
            cs = cs + jnp.sum(mi)
            cb = lax.cond(cs >= _L, lambda: drain(_L, cb, pf), lambda: cb)
            cs = lax.cond(cs >= _L, lambda: cs - _L, lambda: cs)
            return (cs, cb)

        cs, cb = lax.fori_loop(0, ngr, scan_body, (0, cb))
        cb = lax.cond(cs > 0, lambda: drain(cs, cb, pf), lambda: cb)
        pn = jnp.where(p == _NBUF - 1, 0, p + 1)
        return (cb, pn)

    cb, _ = lax.fori_loop(0, n_o, chunk_body, (0, 0))
    lax.cond(cb > 0, lambda: flush(cb), lambda: cb)


@jax.jit
def kernel(input_, weight):
    idx2 = input_.astype(jnp.int32).reshape(_BATCH // 128, 128)
    wt = jnp.swapaxes(weight, 0, 1)
    f = pl.kernel(
        _body,
        mesh=plsc.VectorSubcoreMesh(core_axis_name="c", subcore_axis_name="s"),
        out_type=jax.ShapeDtypeStruct((_BATCH + _NW, 128), jnp.float32),
        scratch_types=[
            pltpu.VMEM((_L, 128), jnp.int32),
            pltpu.VMEM((_BATCH,), jnp.int32),
            pltpu.VMEM((_NBUF, _EMBED_DIM, _CHUNK), jnp.float32),
            pltpu.VMEM((2 * _L,), jnp.int32),
            pltpu.VMEM((2 * _L,), jnp.int32),
            pltpu.VMEM((_RB, 128), jnp.float32),
            pltpu.VMEM((_RB,), jnp.int32),
            pltpu.SemaphoreType.DMA,
        ],
        compiler_params=pltpu.CompilerParams(needs_layout_passes=False),
    )
    out2 = f(idx2, wt)
    return out2[:_BATCH, :_EMBED_DIM]
